# baseline (speedup 1.0000x reference)
import functools
import math

import jax
import jax.numpy as jnp
from jax import lax
from jax.experimental import pallas as pl
from jax.experimental.pallas import tpu as pltpu
from jax.experimental.pallas import tpu_sc as plsc

B = 16384
FEAT = 64
CHUNK = 128


def _gather3(ids, locs, ages, id_table, loc_table, age_table):
    info = plsc.get_sparse_core_info()
    nw = info.num_cores * info.num_subcores
    b_per_w = B // nw
    n_chunks = b_per_w // CHUNK

    ids2 = ids.reshape(B // CHUNK, CHUNK)
    locs2 = locs.reshape(B // CHUNK, CHUNK)
    ages2 = ages.reshape(B // CHUNK, CHUNK)

    mesh = plsc.VectorSubcoreMesh(core_axis_name="c", subcore_axis_name="s")

    @functools.partial(
        pl.kernel,
        mesh=mesh,
        out_type=[jax.ShapeDtypeStruct((B, FEAT), jnp.float32)] * 3,
        scratch_types=(
            [pltpu.VMEM((n_chunks, CHUNK), jnp.int32)] * 3
            + [pltpu.VMEM((b_per_w, FEAT), jnp.float32)] * 3
            + [pltpu.SemaphoreType.DMA] * 3
        ),
        compiler_params=pltpu.CompilerParams(use_tc_tiling_on_sc=False),
    )
    def gather_k(ids_h, locs_h, ages_h, idt_h, loct_h, aget_h,
                 out_id, out_loc, out_age,
                 idx0, idx1, idx2, rows0, rows1, rows2, sem0, sem1, sem2):
        wid = lax.axis_index("s") * info.num_cores + lax.axis_index("c")
        base = wid * b_per_w
        crow = wid * n_chunks
        pltpu.sync_copy(ids_h.at[pl.ds(crow, n_chunks)], idx0)
        pltpu.sync_copy(locs_h.at[pl.ds(crow, n_chunks)], idx1)
        pltpu.sync_copy(ages_h.at[pl.ds(crow, n_chunks)], idx2)
        copies = []
        for j in range(n_chunks):
            dst = pl.ds(j * CHUNK, CHUNK)
            copies.append(pltpu.async_copy(idt_h.at[idx0.at[j]], rows0.at[dst], sem0))
            copies.append(pltpu.async_copy(loct_h.at[idx1.at[j]], rows1.at[dst], sem1))
            copies.append(pltpu.async_copy(aget_h.at[idx2.at[j]], rows2.at[dst], sem2))
        for c in copies:
            c.wait()
        pltpu.sync_copy(rows0, out_id.at[pl.ds(base, b_per_w)])
        pltpu.sync_copy(rows1, out_loc.at[pl.ds(base, b_per_w)])
        pltpu.sync_copy(rows2, out_age.at[pl.ds(base, b_per_w)])

    return gather_k(ids2, locs2, ages2, id_table, loc_table, age_table)


_INV_SQRT2 = 1.0 / math.sqrt(2.0)


def _gelu(x):
    return 0.5 * x * (1.0 + lax.erf(x * _INV_SQRT2))


def _ln(x, eps=1e-5):
    mu = jnp.mean(x, axis=-1, keepdims=True)
    var = jnp.mean((x - mu) * (x - mu), axis=-1, keepdims=True)
    return (x - mu) * lax.rsqrt(var + eps)


def _mlp_body(id_ref, loc_ref, age_ref, w1_ref, b1_ref, w2_ref, b2_ref,
              w3_ref, b3_ref, out_ref):
    w1 = w1_ref[...]
    h = (
        jnp.dot(id_ref[...], w1[0:FEAT], preferred_element_type=jnp.float32)
        + jnp.dot(loc_ref[...], w1[FEAT:2 * FEAT], preferred_element_type=jnp.float32)
        + jnp.dot(age_ref[...], w1[2 * FEAT:3 * FEAT], preferred_element_type=jnp.float32)
        + b1_ref[...]
    )
    h = _gelu(_ln(h))
    h = jnp.dot(h, w2_ref[...], preferred_element_type=jnp.float32) + b2_ref[...]
    h = _gelu(_ln(h))
    h = jnp.dot(h, w3_ref[...], preferred_element_type=jnp.float32) + b3_ref[...]
    out_ref[...] = _gelu(h)


def _mlp(id_emb, loc_emb, age_emb, W1, b1, W2, b2, W3, b3, blk=2048, interpret=False):
    grid = (B // blk,)
    rep = lambda i: (0, 0)
    return pl.pallas_call(
        _mlp_body,
        grid=grid,
        in_specs=[
            pl.BlockSpec((blk, FEAT), lambda i: (i, 0)),
            pl.BlockSpec((blk, FEAT), lambda i: (i, 0)),
            pl.BlockSpec((blk, FEAT), lambda i: (i, 0)),
            pl.BlockSpec((3 * FEAT, 128), rep),
            pl.BlockSpec((1, 128), rep),
            pl.BlockSpec((128, 64), rep),
            pl.BlockSpec((1, 64), rep),
            pl.BlockSpec((64, 128), rep),
            pl.BlockSpec((1, 128), rep),
        ],
        out_specs=pl.BlockSpec((blk, 128), lambda i: (i, 0)),
        out_shape=jax.ShapeDtypeStruct((B, 128), jnp.float32),
        interpret=interpret,
    )(id_emb, loc_emb, age_emb, W1, b1.reshape(1, -1), W2, b2.reshape(1, -1),
      W3, b3.reshape(1, -1))


def kernel(user_ids, user_locations, user_ages, id_table, loc_table, age_table,
           W1, b1, W2, b2, W3, b3):
    ids = user_ids.astype(jnp.int32)
    locs = user_locations.astype(jnp.int32)
    ages = user_ages.astype(jnp.int32)
    id_emb, loc_emb, age_emb = _gather3(ids, locs, ages, id_table, loc_table,
                                        age_table)
    return _mlp(id_emb, loc_emb, age_emb, W1, b1, W2, b2, W3, b3)
